# Initial kernel scaffold; baseline (speedup 1.0000x reference)
import jax
import jax.numpy as jnp
from jax.experimental import pallas as pl


def kernel(h, x, edge_attr, We1, be1, We2, be2, Wc1, bc1, Wc2, bc2, Wn1, bn1, Wn2, bn2, edge_index):
    raise NotImplementedError("write your pallas kernel here")


# speedup vs baseline: 2.1297x; 2.1297x over previous
import jax
import jax.numpy as jnp
from jax import lax
from jax.experimental import pallas as pl
from jax.experimental.pallas import tpu as pltpu
from jax.experimental.pallas import tpu_sc as plsc

N = 10000
E = 320000
D = 128
ED = 16

NT = 32
CG = 256
EPT = 10240
E_PAD = NT * EPT
NCH_G = EPT // CG
CS = 128
NCH_S = EPT // CS
N_PAD = 10240
NXR = N_PAD // 32
N_X = 10016

EB = 512
NE_BLK = E_PAD // EB
NB = 400

_F32 = jnp.float32


def _pre_body(h_ref, wa_ref, wb_ref, be1_ref, a_ref, b_ref):
    hb = h_ref[...]
    a_ref[...] = jnp.dot(hb, wa_ref[...], preferred_element_type=_F32) \
        + be1_ref[...]
    b_ref[...] = jnp.dot(hb, wb_ref[...], preferred_element_type=_F32)


def _build_tables(h, wa, wb, be1):
    return pl.pallas_call(
        _pre_body,
        grid=(N // NB,),
        in_specs=[
            pl.BlockSpec((NB, D), lambda i: (i, 0)),
            pl.BlockSpec((D, D), lambda i: (0, 0)),
            pl.BlockSpec((D, D), lambda i: (0, 0)),
            pl.BlockSpec((1, D), lambda i: (0, 0)),
        ],
        out_specs=[
            pl.BlockSpec((NB, D), lambda i: (i, 0)),
            pl.BlockSpec((NB, D), lambda i: (i, 0)),
        ],
        out_shape=[
            jax.ShapeDtypeStruct((N, D), _F32),
            jax.ShapeDtypeStruct((N, D), _F32),
        ],
    )(h, wa, wb, be1)


def _gather_body(a_hbm, b_hbm, xf_hbm, wrad_hbm, row_hbm, col_hbm, out_hbm,
                 xref, wradv, ridx, cidx, rbuf, abuf, bbuf, sem):
    c = lax.axis_index("c")
    s = lax.axis_index("s")
    wid = s * 2 + c
    pltpu.sync_copy(xf_hbm, xref)
    pltpu.sync_copy(wrad_hbm, wradv)

    def chunk(j, carry):
        base = wid * EPT + j * CG
        pltpu.sync_copy(row_hbm.at[pl.ds(base, CG)], ridx)
        pltpu.sync_copy(col_hbm.at[pl.ds(base, CG)], cidx)
        cps = []
        for k in range(CG // 128):
            sl = pl.ds(k * 128, 128)
            cps.append(pltpu.async_copy(a_hbm.at[ridx.at[sl]], abuf.at[sl],
                                        sem))
            cps.append(pltpu.async_copy(b_hbm.at[cidx.at[sl]], bbuf.at[sl],
                                        sem))
        for g in range(CG // 16):
            sl16 = pl.ds(g * 16, 16)
            rv4 = ridx[sl16] * 4
            cv4 = cidx[sl16] * 4
            rad = jnp.zeros((16,), _F32)
            for j2 in range(3):
                cdj = plsc.load_gather(xref, [rv4 + j2]) \
                    - plsc.load_gather(xref, [cv4 + j2])
                rad = rad + cdj * cdj
            rbuf[sl16] = rad
        for cp in cps:
            cp.wait()

        def rowfn(g, carry2):
            rad16 = rbuf[pl.ds(g * 16, 16)]
            for r in range(16):
                i = g * 16 + r
                rad = rad16[r]
                for jb in range(8):
                    sl2 = pl.ds(jb * 16, 16)
                    abuf[i, sl2] = abuf[i, sl2] + bbuf[i, sl2] \
                        + rad * wradv[sl2]
            return carry2

        lax.fori_loop(0, CG // 16, rowfn, 0)
        pltpu.sync_copy(abuf, out_hbm.at[pl.ds(base, CG)])
        return carry

    lax.fori_loop(0, NCH_G, chunk, 0)


def _sc_gather(a_tab, b_tab, xflat, wrad1, row_g, col_g):
    mesh = plsc.VectorSubcoreMesh(core_axis_name="c", subcore_axis_name="s")
    return pl.kernel(
        _gather_body,
        out_type=jax.ShapeDtypeStruct((E_PAD, D), _F32),
        mesh=mesh,
        scratch_types=[
            pltpu.VMEM((4 * N_X,), _F32),
            pltpu.VMEM((D,), _F32),
            pltpu.VMEM((CG,), jnp.int32),
            pltpu.VMEM((CG,), jnp.int32),
            pltpu.VMEM((CG,), _F32),
            pltpu.VMEM((CG, D), _F32),
            pltpu.VMEM((CG, D), _F32),
            pltpu.SemaphoreType.DMA,
        ],
        compiler_params=pltpu.CompilerParams(needs_layout_passes=False),
    )(a_tab, b_tab, xflat, wrad1, row_g, col_g)


def _edge_body(g_ref, ea_ref, wea_ref, we2_ref, be2_ref,
               wc1_ref, bc1_ref, wc2r_ref, bc2_ref, mij_ref, fs_ref):
    pre = g_ref[...] + jnp.dot(ea_ref[...], wea_ref[...],
                               preferred_element_type=_F32)
    m = jax.nn.silu(pre)
    mij = jax.nn.silu(jnp.dot(m, we2_ref[...], preferred_element_type=_F32)
                      + be2_ref[...])
    t = jax.nn.silu(jnp.dot(mij, wc1_ref[...], preferred_element_type=_F32)
                    + bc1_ref[...])
    fst = lax.dot_general(wc2r_ref[...], t, (((1,), (1,)), ((), ())),
                          preferred_element_type=_F32) + bc2_ref[...]
    mij_ref[...] = mij
    fs_ref[...] = fst.reshape(1, 1, EB)


def _edge_mlp(g2, ea_pad, wea, we2, be2, wc1, bc1, wc2r, bc2):
    wspec = pl.BlockSpec((D, D), lambda i: (0, 0))
    bspec = pl.BlockSpec((1, D), lambda i: (0, 0))
    return pl.pallas_call(
        _edge_body,
        grid=(NE_BLK,),
        in_specs=[
            pl.BlockSpec((EB, D), lambda i: (i, 0)),
            pl.BlockSpec((EB, ED), lambda i: (i, 0)),
            pl.BlockSpec((ED, D), lambda i: (0, 0)),
            wspec, bspec, wspec, bspec, bspec,
            pl.BlockSpec((1, 1), lambda i: (0, 0)),
        ],
        out_specs=[
            pl.BlockSpec((EB, D), lambda i: (i, 0)),
            pl.BlockSpec((1, 1, EB), lambda i: (i, 0, 0)),
        ],
        out_shape=[
            jax.ShapeDtypeStruct((E_PAD, D), _F32),
            jax.ShapeDtypeStruct((NE_BLK, 1, EB), _F32),
        ],
    )(g2, ea_pad, wea, we2, be2, wc1, bc1, wc2r, bc2)


def _scatter_h_body(m_hbm, row_hbm, outh_hbm, ridx, mbuf, acc_h):
    c = lax.axis_index("c")
    s = lax.axis_index("s")
    wid = s * 2 + c

    def zrow(i, carry):
        for jb in range(8):
            mbuf[i, pl.ds(jb * 16, 16)] = jnp.zeros((16,), _F32)
        return carry

    lax.fori_loop(0, CS, zrow, 0)
    for k in range(N_PAD // 16 // CS):
        pltpu.sync_copy(mbuf, acc_h.at[pl.ds(s * (N_PAD // 16) + k * CS, CS)])
    plsc.subcore_barrier()

    def chunk(j, carry):
        base = wid * EPT + j * CS
        pltpu.sync_copy(row_hbm.at[pl.ds(base, CS)], ridx)
        pltpu.sync_copy(m_hbm.at[pl.ds(base, CS)], mbuf)
        pltpu.sync_copy(mbuf, acc_h.at[ridx], add=True)
        return carry

    lax.fori_loop(0, NCH_S, chunk, 0)
    plsc.subcore_barrier()
    pltpu.sync_copy(acc_h.at[pl.ds(s * (N_PAD // 16), N_PAD // 16)],
                    outh_hbm.at[c].at[pl.ds(s * (N_PAD // 16), N_PAD // 16)])


def _sc_scatter_h(mij, row_s):
    mesh = plsc.VectorSubcoreMesh(core_axis_name="c", subcore_axis_name="s")
    return pl.kernel(
        _scatter_h_body,
        out_type=jax.ShapeDtypeStruct((2, N_PAD, D), _F32),
        mesh=mesh,
        scratch_types=[
            pltpu.VMEM((CS,), jnp.int32),
            pltpu.VMEM((CS, D), _F32),
            pltpu.VMEM_SHARED((N_PAD, D), _F32),
        ],
        compiler_params=pltpu.CompilerParams(needs_layout_passes=False),
    )(mij, row_s)


def _scatter_x_body(fs_hbm, xf_hbm, row_hbm, col_hbm, outx_hbm,
                    xref, ridx, cidx, xidx, fsbuf, fvbuf, acc_x):
    c = lax.axis_index("c")
    s = lax.axis_index("s")
    wid = s * 2 + c

    def zrow(i, carry):
        for jb in range(8):
            fvbuf[i, pl.ds(jb * 16, 16)] = jnp.zeros((16,), _F32)
        return carry

    lax.fori_loop(0, CS, zrow, 0)

    @pl.when(s < NXR // 32)
    def _():
        pltpu.sync_copy(fvbuf.at[pl.ds(0, 32)], acc_x.at[pl.ds(s * 32, 32)])

    plsc.subcore_barrier()
    pltpu.sync_copy(xf_hbm, xref)
    lanes = lax.iota(jnp.int32, 16)

    def chunk(j, carry):
        base = wid * EPT + j * CS
        pltpu.sync_copy(row_hbm.at[pl.ds(base, CS)], ridx)
        pltpu.sync_copy(col_hbm.at[pl.ds(base, CS)], cidx)
        pltpu.sync_copy(fs_hbm.at[pl.ds(base, CS)], fsbuf)
        for g in range(CS // 16):
            sl16 = pl.ds(g * 16, 16)
            rv = ridx[sl16]
            rv4 = rv * 4
            cv4 = cidx[sl16] * 4
            fsv = fsbuf[sl16]
            evec = lanes + (g * 16)
            lane0 = (rv & 31) * 4
            for j2 in range(3):
                cdj = plsc.load_gather(xref, [rv4 + j2]) \
                    - plsc.load_gather(xref, [cv4 + j2])
                plsc.store_scatter(fvbuf, [evec, lane0 + j2], cdj * fsv)
            xidx[sl16] = lax.shift_right_logical(rv, 5)
        pltpu.sync_copy(fvbuf, acc_x.at[xidx], add=True)
        zv = jnp.zeros((16,), _F32)
        for g in range(CS // 16):
            sl16 = pl.ds(g * 16, 16)
            lane0 = (ridx[sl16] & 31) * 4
            evec = lanes + (g * 16)
            for j2 in range(3):
                plsc.store_scatter(fvbuf, [evec, lane0 + j2], zv)
        return carry

    lax.fori_loop(0, NCH_S, chunk, 0)
    plsc.subcore_barrier()

    @pl.when(s < NXR // 32)
    def _():
        pltpu.sync_copy(acc_x.at[pl.ds(s * 32, 32)],
                        outx_hbm.at[c].at[pl.ds(s * 32, 32)])


def _sc_scatter_x(fs1, xflat, row_s, col_g):
    mesh = plsc.VectorSubcoreMesh(core_axis_name="c", subcore_axis_name="s")
    return pl.kernel(
        _scatter_x_body,
        out_type=jax.ShapeDtypeStruct((2, NXR, D), _F32),
        mesh=mesh,
        scratch_types=[
            pltpu.VMEM((4 * N_X,), _F32),
            pltpu.VMEM((CS,), jnp.int32),
            pltpu.VMEM((CS,), jnp.int32),
            pltpu.VMEM((CS,), jnp.int32),
            pltpu.VMEM((CS,), _F32),
            pltpu.VMEM((CS, D), _F32),
            pltpu.VMEM_SHARED((NXR, D), _F32),
        ],
        compiler_params=pltpu.CompilerParams(needs_layout_passes=False),
    )(fs1, xflat, row_s, col_g)


def _node_body(h_ref, x4_ref, p0_ref, p1_ref, q0_ref, q1_ref,
               wn1h_ref, wn1m_ref, bn1_ref, wn2_ref, bn2_ref,
               hn_ref, xn_ref):
    hb = h_ref[...]
    mi = p0_ref[...] + p1_ref[...]
    xu = q0_ref[...] + q1_ref[...]
    u = jax.nn.silu(jnp.dot(hb, wn1h_ref[...], preferred_element_type=_F32)
                    + jnp.dot(mi, wn1m_ref[...], preferred_element_type=_F32)
                    + bn1_ref[...])
    hn_ref[...] = hb + jnp.dot(u, wn2_ref[...], preferred_element_type=_F32) \
        + bn2_ref[...]
    xn_ref[...] = x4_ref[...] + xu


def _node_mlp(h, x4, p0, p1, q0, q1, wn1h, wn1m, bn1, wn2, bn2):
    wspec = pl.BlockSpec((D, D), lambda i: (0, 0))
    bspec = pl.BlockSpec((1, D), lambda i: (0, 0))
    return pl.pallas_call(
        _node_body,
        grid=(N // NB,),
        in_specs=[
            pl.BlockSpec((NB, D), lambda i: (i, 0)),
            pl.BlockSpec((NB, 4), lambda i: (i, 0)),
            pl.BlockSpec((NB, D), lambda i: (i, 0)),
            pl.BlockSpec((NB, D), lambda i: (i, 0)),
            pl.BlockSpec((NB, 4), lambda i: (i, 0)),
            pl.BlockSpec((NB, 4), lambda i: (i, 0)),
            wspec, wspec, bspec, wspec, bspec,
        ],
        out_specs=[
            pl.BlockSpec((NB, D), lambda i: (i, 0)),
            pl.BlockSpec((NB, 4), lambda i: (i, 0)),
        ],
        out_shape=[
            jax.ShapeDtypeStruct((N, D), _F32),
            jax.ShapeDtypeStruct((N, 4), _F32),
        ],
    )(h, x4, p0, p1, q0, q1, wn1h, wn1m, bn1, wn2, bn2)


def kernel(h, x, edge_attr, We1, be1, We2, be2, Wc1, bc1, Wc2, bc2,
           Wn1, bn1, Wn2, bn2, edge_index):
    row = edge_index[0]
    col = edge_index[1]
    x4 = jnp.pad(x, ((0, 0), (0, 1)))
    xflat = jnp.pad(x, ((0, N_X - N), (0, 1))).reshape(-1)

    wa = We1[:D]
    wb = We1[D:2 * D]
    wrad1 = We1[2 * D]
    wea = We1[2 * D + 1:]
    be1r = be1.reshape(1, D)
    be2r = be2.reshape(1, D)
    bc1r = bc1.reshape(1, D)
    wc2r = Wc2.reshape(1, D)
    bc2r = bc2.reshape(1, 1)
    wn1h = Wn1[:D]
    wn1m = Wn1[D:]
    bn1r = bn1.reshape(1, D)
    bn2r = bn2.reshape(1, D)

    pad = E_PAD - E
    row_g = jnp.concatenate([row, jnp.zeros((pad,), jnp.int32)])
    col_g = jnp.concatenate([col, jnp.zeros((pad,), jnp.int32)])
    row_s = jnp.concatenate([row, jnp.full((pad,), N, jnp.int32)])
    ea_pad = jnp.concatenate([edge_attr, jnp.zeros((pad, ED), _F32)])

    a_tab, b_tab = _build_tables(h, wa, wb, be1r)
    g2 = _sc_gather(a_tab, b_tab, xflat, wrad1, row_g, col_g)
    mij, fs2 = _edge_mlp(g2, ea_pad, wea, We2, be2r, Wc1, bc1r, wc2r, bc2r)
    parts_h = _sc_scatter_h(mij, row_s)
    parts_x = _sc_scatter_x(fs2.reshape(E_PAD), xflat, row_s, col_g)
    q0 = parts_x[0].reshape(N_PAD, 4)
    q1 = parts_x[1].reshape(N_PAD, 4)
    h_new, xn4 = _node_mlp(h, x4, parts_h[0], parts_h[1], q0, q1,
                           wn1h, wn1m, bn1r, Wn2, bn2r)
    return (h_new, xn4[:, :3])


# speedup vs baseline: 2.7766x; 1.3037x over previous
import jax
import jax.numpy as jnp
from jax import lax
from jax.experimental import pallas as pl
from jax.experimental.pallas import tpu as pltpu
from jax.experimental.pallas import tpu_sc as plsc

N = 10000
E = 320000
D = 128
ED = 16

NT = 32
CH = 128
EPT = 10240
E_PAD = NT * EPT
NCH = EPT // CH
N_PAD = 10240
NXR = N_PAD // 32

EB = 512
NE_BLK = E_PAD // EB
NB = 400
NBP = 512

_F32 = jnp.float32


def _pre_body(h_ref, wa_ref, wb_ref, be1_ref, a_ref, b_ref):
    hb = h_ref[...]
    a_ref[...] = jnp.dot(hb, wa_ref[...], preferred_element_type=_F32) \
        + be1_ref[...]
    b_ref[...] = jnp.dot(hb, wb_ref[...], preferred_element_type=_F32)


def _build_tables(h_pad, wa, wb, be1):
    return pl.pallas_call(
        _pre_body,
        grid=(N_PAD // NBP,),
        in_specs=[
            pl.BlockSpec((NBP, D), lambda i: (i, 0)),
            pl.BlockSpec((D, D), lambda i: (0, 0)),
            pl.BlockSpec((D, D), lambda i: (0, 0)),
            pl.BlockSpec((1, D), lambda i: (0, 0)),
        ],
        out_specs=[
            pl.BlockSpec((NBP, D), lambda i: (i, 0)),
            pl.BlockSpec((NBP, D), lambda i: (i, 0)),
        ],
        out_shape=[
            jax.ShapeDtypeStruct((N_PAD, D), _F32),
            jax.ShapeDtypeStruct((N_PAD, D), _F32),
        ],
    )(h_pad, wa, wb, be1)


def _gather_body(a_hbm, b_hbm, xf_hbm, wrad_hbm, rc_hbm, out_hbm,
                 xref, wradv, idx0, idx1, abuf0, bbuf0, abuf1, bbuf1, rbuf,
                 gsem0, gsem1, wsem0, wsem1):
    c = lax.axis_index("c")
    s = lax.axis_index("s")
    wid = s * 2 + c
    pltpu.sync_copy(xf_hbm, xref)
    pltpu.sync_copy(wrad_hbm, wradv)
    w8 = [wradv[pl.ds(jb * 16, 16)] for jb in range(8)]
    idxb = (idx0, idx1)
    abufs = (abuf0, abuf1)
    bbufs = (bbuf0, bbuf1)
    gsems = (gsem0, gsem1)
    wsems = (wsem0, wsem1)

    def issue(j, p):
        pltpu.sync_copy(rc_hbm.at[wid].at[j], idxb[p])
        pltpu.async_copy(a_hbm.at[idxb[p].at[0]], abufs[p], gsems[p])
        pltpu.async_copy(b_hbm.at[idxb[p].at[1]], bbufs[p], gsems[p])

    issue(0, 0)

    def outer(jo, carry):
        j2 = jo * 2
        for p in (0, 1):
            j = j2 + p
            q = 1 - p
            ab = abufs[p]
            bb = bbufs[p]

            @pl.when(j >= 1)
            def _():
                pltpu.make_async_copy(abufs[q], out_hbm.at[pl.ds(0, CH)],
                                      wsems[q]).wait()

            @pl.when(j + 1 < NCH)
            def _():
                issue(j + 1, q)

            for g in range(CH // 16):
                sl16 = pl.ds(g * 16, 16)
                rv4 = idxb[p][0, sl16] * 4
                cv4 = idxb[p][1, sl16] * 4
                rad = jnp.zeros((16,), _F32)
                for j2c in range(3):
                    cdj = plsc.load_gather(xref, [rv4 + j2c]) \
                        - plsc.load_gather(xref, [cv4 + j2c])
                    rad = rad + cdj * cdj
                rbuf[sl16] = rad

            pltpu.make_async_copy(a_hbm.at[pl.ds(0, CH)], ab, gsems[p]).wait()
            pltpu.make_async_copy(a_hbm.at[pl.ds(0, CH)], bb, gsems[p]).wait()

            def rowfn(g, carry2):
                rad16 = rbuf[pl.ds(g * 16, 16)]
                for r in range(16):
                    i = g * 16 + r
                    rad = rad16[r]
                    for jb in range(8):
                        sl2 = pl.ds(jb * 16, 16)
                        ab[i, sl2] = ab[i, sl2] + bb[i, sl2] + rad * w8[jb]
                return carry2

            lax.fori_loop(0, CH // 16, rowfn, 0)
            base = wid * EPT + j * CH
            pltpu.async_copy(ab, out_hbm.at[pl.ds(base, CH)], wsems[p])
        return carry

    lax.fori_loop(0, NCH // 2, outer, 0)
    pltpu.make_async_copy(abuf1, out_hbm.at[pl.ds(0, CH)], wsem1).wait()


def _sc_gather(a_tab, b_tab, xflat, wrad1, rc4):
    mesh = plsc.VectorSubcoreMesh(core_axis_name="c", subcore_axis_name="s")
    return pl.kernel(
        _gather_body,
        out_type=jax.ShapeDtypeStruct((E_PAD, D), _F32),
        mesh=mesh,
        scratch_types=[
            pltpu.VMEM((4 * N_PAD,), _F32),
            pltpu.VMEM((D,), _F32),
            pltpu.VMEM((2, CH), jnp.int32),
            pltpu.VMEM((2, CH), jnp.int32),
            pltpu.VMEM((CH, D), _F32),
            pltpu.VMEM((CH, D), _F32),
            pltpu.VMEM((CH, D), _F32),
            pltpu.VMEM((CH, D), _F32),
            pltpu.VMEM((CH,), _F32),
            pltpu.SemaphoreType.DMA,
            pltpu.SemaphoreType.DMA,
            pltpu.SemaphoreType.DMA,
            pltpu.SemaphoreType.DMA,
        ],
        compiler_params=pltpu.CompilerParams(needs_layout_passes=False),
    )(a_tab, b_tab, xflat, wrad1, rc4)


def _edge_body(g_ref, ea_ref, wea_ref, we2_ref, be2_ref,
               wc1_ref, bc1_ref, wc2r_ref, bc2_ref, mij_ref, fs_ref):
    pre = g_ref[...] + jnp.dot(ea_ref[...], wea_ref[...],
                               preferred_element_type=_F32)
    m = jax.nn.silu(pre)
    mij = jax.nn.silu(jnp.dot(m, we2_ref[...], preferred_element_type=_F32)
                      + be2_ref[...])
    t = jax.nn.silu(jnp.dot(mij, wc1_ref[...], preferred_element_type=_F32)
                    + bc1_ref[...])
    fst = lax.dot_general(wc2r_ref[...], t, (((1,), (1,)), ((), ())),
                          preferred_element_type=_F32) + bc2_ref[...]
    mij_ref[...] = mij
    fs_ref[...] = fst.reshape(1, 1, EB)


def _edge_mlp(g2, ea_pad, wea, we2, be2, wc1, bc1, wc2r, bc2):
    wspec = pl.BlockSpec((D, D), lambda i: (0, 0))
    bspec = pl.BlockSpec((1, D), lambda i: (0, 0))
    return pl.pallas_call(
        _edge_body,
        grid=(NE_BLK,),
        in_specs=[
            pl.BlockSpec((EB, D), lambda i: (i, 0)),
            pl.BlockSpec((EB, ED), lambda i: (i, 0)),
            pl.BlockSpec((ED, D), lambda i: (0, 0)),
            wspec, bspec, wspec, bspec, bspec,
            pl.BlockSpec((1, 1), lambda i: (0, 0)),
        ],
        out_specs=[
            pl.BlockSpec((EB, D), lambda i: (i, 0)),
            pl.BlockSpec((1, 1, EB), lambda i: (i, 0, 0)),
        ],
        out_shape=[
            jax.ShapeDtypeStruct((E_PAD, D), _F32),
            jax.ShapeDtypeStruct((NE_BLK, 1, EB), _F32),
        ],
    )(g2, ea_pad, wea, we2, be2, wc1, bc1, wc2r, bc2)


def _scatter_h_body(m_hbm, rc_hbm, outh_hbm, idxb, mbuf, acc_h):
    c = lax.axis_index("c")
    s = lax.axis_index("s")
    wid = s * 2 + c

    def zrow(i, carry):
        for jb in range(8):
            mbuf[i, pl.ds(jb * 16, 16)] = jnp.zeros((16,), _F32)
        return carry

    lax.fori_loop(0, CH, zrow, 0)
    for k in range(N_PAD // 16 // CH):
        pltpu.sync_copy(mbuf, acc_h.at[pl.ds(s * (N_PAD // 16) + k * CH, CH)])
    plsc.subcore_barrier()

    def chunk(j, carry):
        base = wid * EPT + j * CH
        pltpu.sync_copy(rc_hbm.at[wid].at[j], idxb)
        pltpu.sync_copy(m_hbm.at[pl.ds(base, CH)], mbuf)
        pltpu.sync_copy(mbuf, acc_h.at[idxb.at[0]], add=True)
        return carry

    lax.fori_loop(0, NCH, chunk, 0)
    plsc.subcore_barrier()
    pltpu.sync_copy(acc_h.at[pl.ds(s * (N_PAD // 16), N_PAD // 16)],
                    outh_hbm.at[c].at[pl.ds(s * (N_PAD // 16), N_PAD // 16)])


def _sc_scatter_h(mij, rc4):
    mesh = plsc.VectorSubcoreMesh(core_axis_name="c", subcore_axis_name="s")
    return pl.kernel(
        _scatter_h_body,
        out_type=jax.ShapeDtypeStruct((2, N_PAD, D), _F32),
        mesh=mesh,
        scratch_types=[
            pltpu.VMEM((2, CH), jnp.int32),
            pltpu.VMEM((CH, D), _F32),
            pltpu.VMEM_SHARED((N_PAD, D), _F32),
        ],
        compiler_params=pltpu.CompilerParams(needs_layout_passes=False),
    )(mij, rc4)


def _scatter_x_body(fs_hbm, xf_hbm, rc_hbm, outx_hbm,
                    xref, idxb, xidx, fsbuf, fvbuf, acc_x):
    c = lax.axis_index("c")
    s = lax.axis_index("s")
    wid = s * 2 + c

    def zrow(i, carry):
        for jb in range(8):
            fvbuf[i, pl.ds(jb * 16, 16)] = jnp.zeros((16,), _F32)
        return carry

    lax.fori_loop(0, CH, zrow, 0)

    @pl.when(s < NXR // 32)
    def _():
        pltpu.sync_copy(fvbuf.at[pl.ds(0, 32)], acc_x.at[pl.ds(s * 32, 32)])

    plsc.subcore_barrier()
    pltpu.sync_copy(xf_hbm, xref)
    lanes = lax.iota(jnp.int32, 16)

    def chunk(j, carry):
        base = wid * EPT + j * CH
        pltpu.sync_copy(rc_hbm.at[wid].at[j], idxb)
        pltpu.sync_copy(fs_hbm.at[pl.ds(base, CH)], fsbuf)
        for g in range(CH // 16):
            sl16 = pl.ds(g * 16, 16)
            rv = idxb[0, sl16]
            rv4 = rv * 4
            cv4 = idxb[1, sl16] * 4
            fsv = fsbuf[sl16]
            evec = lanes + (g * 16)
            lane0 = (rv & 31) * 4
            for j2 in range(3):
                cdj = plsc.load_gather(xref, [rv4 + j2]) \
                    - plsc.load_gather(xref, [cv4 + j2])
                plsc.store_scatter(fvbuf, [evec, lane0 + j2], cdj * fsv)
            xidx[sl16] = lax.shift_right_logical(rv, 5)
        pltpu.sync_copy(fvbuf, acc_x.at[xidx], add=True)
        zv = jnp.zeros((16,), _F32)
        for g in range(CH // 16):
            sl16 = pl.ds(g * 16, 16)
            lane0 = (idxb[0, sl16] & 31) * 4
            evec = lanes + (g * 16)
            for j2 in range(3):
                plsc.store_scatter(fvbuf, [evec, lane0 + j2], zv)
        return carry

    lax.fori_loop(0, NCH, chunk, 0)
    plsc.subcore_barrier()

    @pl.when(s < NXR // 32)
    def _():
        pltpu.sync_copy(acc_x.at[pl.ds(s * 32, 32)],
                        outx_hbm.at[c].at[pl.ds(s * 32, 32)])


def _sc_scatter_x(fs1, xflat, rc4):
    mesh = plsc.VectorSubcoreMesh(core_axis_name="c", subcore_axis_name="s")
    return pl.kernel(
        _scatter_x_body,
        out_type=jax.ShapeDtypeStruct((2, NXR, D), _F32),
        mesh=mesh,
        scratch_types=[
            pltpu.VMEM((4 * N_PAD,), _F32),
            pltpu.VMEM((2, CH), jnp.int32),
            pltpu.VMEM((CH,), jnp.int32),
            pltpu.VMEM((CH,), _F32),
            pltpu.VMEM((CH, D), _F32),
            pltpu.VMEM_SHARED((NXR, D), _F32),
        ],
        compiler_params=pltpu.CompilerParams(needs_layout_passes=False),
    )(fs1, xflat, rc4)


def _node_body(h_ref, x4_ref, p0_ref, p1_ref, q0_ref, q1_ref,
               wn1h_ref, wn1m_ref, bn1_ref, wn2_ref, bn2_ref,
               hn_ref, xn_ref):
    hb = h_ref[...]
    mi = p0_ref[...] + p1_ref[...]
    xu = q0_ref[...] + q1_ref[...]
    u = jax.nn.silu(jnp.dot(hb, wn1h_ref[...], preferred_element_type=_F32)
                    + jnp.dot(mi, wn1m_ref[...], preferred_element_type=_F32)
                    + bn1_ref[...])
    hn_ref[...] = hb + jnp.dot(u, wn2_ref[...], preferred_element_type=_F32) \
        + bn2_ref[...]
    xn_ref[...] = x4_ref[...] + xu


def _node_mlp(h, x4, p0, p1, q0, q1, wn1h, wn1m, bn1, wn2, bn2):
    wspec = pl.BlockSpec((D, D), lambda i: (0, 0))
    bspec = pl.BlockSpec((1, D), lambda i: (0, 0))
    return pl.pallas_call(
        _node_body,
        grid=(N // NB,),
        in_specs=[
            pl.BlockSpec((NB, D), lambda i: (i, 0)),
            pl.BlockSpec((NB, 4), lambda i: (i, 0)),
            pl.BlockSpec((NB, D), lambda i: (i, 0)),
            pl.BlockSpec((NB, D), lambda i: (i, 0)),
            pl.BlockSpec((NB, 4), lambda i: (i, 0)),
            pl.BlockSpec((NB, 4), lambda i: (i, 0)),
            wspec, wspec, bspec, wspec, bspec,
        ],
        out_specs=[
            pl.BlockSpec((NB, D), lambda i: (i, 0)),
            pl.BlockSpec((NB, 4), lambda i: (i, 0)),
        ],
        out_shape=[
            jax.ShapeDtypeStruct((N, D), _F32),
            jax.ShapeDtypeStruct((N, 4), _F32),
        ],
    )(h, x4, p0, p1, q0, q1, wn1h, wn1m, bn1, wn2, bn2)


def kernel(h, x, edge_attr, We1, be1, We2, be2, Wc1, bc1, Wc2, bc2,
           Wn1, bn1, Wn2, bn2, edge_index):
    row = edge_index[0]
    col = edge_index[1]
    x4 = jnp.pad(x, ((0, 0), (0, 1)))
    xflat = jnp.pad(x, ((0, N_PAD - N), (0, 1))).reshape(-1)
    h_pad = jnp.pad(h, ((0, N_PAD - N), (0, 0)))

    wa = We1[:D]
    wb = We1[D:2 * D]
    wrad1 = We1[2 * D]
    wea = We1[2 * D + 1:]
    be1r = be1.reshape(1, D)
    be2r = be2.reshape(1, D)
    bc1r = bc1.reshape(1, D)
    wc2r = Wc2.reshape(1, D)
    bc2r = bc2.reshape(1, 1)
    wn1h = Wn1[:D]
    wn1m = Wn1[D:]
    bn1r = bn1.reshape(1, D)
    bn2r = bn2.reshape(1, D)

    pad = E_PAD - E
    row_p = jnp.concatenate([row, jnp.full((pad,), N, jnp.int32)])
    col_p = jnp.concatenate([col, jnp.full((pad,), N, jnp.int32)])
    rc4 = jnp.stack([row_p.reshape(NT, NCH, CH),
                     col_p.reshape(NT, NCH, CH)], axis=2)
    ea_pad = jnp.concatenate([edge_attr, jnp.zeros((pad, ED), _F32)])

    a_tab, b_tab = _build_tables(h_pad, wa, wb, be1r)
    g2 = _sc_gather(a_tab, b_tab, xflat, wrad1, rc4)
    mij, fs2 = _edge_mlp(g2, ea_pad, wea, We2, be2r, Wc1, bc1r, wc2r, bc2r)
    parts_h = _sc_scatter_h(mij, rc4)
    parts_x = _sc_scatter_x(fs2.reshape(E_PAD), xflat, rc4)
    q0 = parts_x[0].reshape(N_PAD, 4)
    q1 = parts_x[1].reshape(N_PAD, 4)
    h_new, xn4 = _node_mlp(h, x4, parts_h[0], parts_h[1], q0, q1,
                           wn1h, wn1m, bn1r, Wn2, bn2r)
    return (h_new, xn4[:, :3])
